# baseline (speedup 1.0000x reference)
import functools
import jax
import jax.numpy as jnp
from jax import lax
from jax.experimental import pallas as pl
from jax.experimental.pallas import tpu as pltpu
from jax.experimental.pallas import tpu_sc as plsc

_NUM_CORES = 2
_NUM_SUBCORES = 16
_NUM_WORKERS = _NUM_CORES * _NUM_SUBCORES
_LANES = 16
_CHUNK = 128


@functools.lru_cache(maxsize=None)
def _build(B, D):
    b_per_w = B // _NUM_WORKERS
    n_chunks = b_per_w // _CHUNK
    n_groups = b_per_w // _LANES
    mesh = plsc.VectorSubcoreMesh(
        core_axis_name="c", subcore_axis_name="s",
        num_cores=_NUM_CORES, num_subcores=_NUM_SUBCORES,
    )

    @functools.partial(
        pl.kernel,
        out_type=jax.ShapeDtypeStruct((B, D), jnp.float32),
        mesh=mesh,
        scratch_types=[
            pltpu.VMEM((b_per_w * 3,), jnp.int32),
            pltpu.VMEM((b_per_w,), jnp.int32),
            pltpu.VMEM((b_per_w,), jnp.int32),
            pltpu.VMEM((b_per_w,), jnp.int32),
            pltpu.VMEM((b_per_w,), jnp.int32),
            pltpu.VMEM((b_per_w,), jnp.int32),
            pltpu.VMEM((b_per_w,), jnp.int32),
            pltpu.VMEM((_CHUNK, 2 * D), jnp.float32),
            pltpu.VMEM((_CHUNK, 2 * D), jnp.float32),
            pltpu.VMEM((_CHUNK, 2 * D), jnp.float32),
            pltpu.VMEM((b_per_w, D), jnp.float32),
            pltpu.SemaphoreType.DMA,
        ],
        compiler_params=pltpu.CompilerParams(
            use_tc_tiling_on_sc=False, needs_layout_passes=False),
    )
    def dist_mult(tflat_hbm, emb2_hbm, rel2_hbm, out_hbm,
                  trip_v, row_s, row_r, row_o, off_s, off_r, off_o,
                  rows_s, rows_r, rows_o, out_v, sem):
        wid = lax.axis_index("s") * _NUM_CORES + lax.axis_index("c")
        base = wid * b_per_w

        pltpu.sync_copy(tflat_hbm.at[pl.ds(base * 3, b_per_w * 3)], trip_v)

        lanes3 = lax.iota(jnp.int32, 16) * 3

        def idx_body(g, _):
            offs = lanes3 + g * (3 * _LANES)
            gs = pl.ds(g * _LANES, _LANES)
            s = plsc.load_gather(trip_v, [offs])
            r = plsc.load_gather(trip_v, [offs + 1])
            o = plsc.load_gather(trip_v, [offs + 2])
            row_s[gs] = lax.shift_right_logical(s, 1)
            row_r[gs] = lax.shift_right_logical(r, 1)
            row_o[gs] = lax.shift_right_logical(o, 1)
            off_s[gs] = lax.shift_left(lax.bitwise_and(s, 1), 6)
            off_r[gs] = lax.shift_left(lax.bitwise_and(r, 1), 6)
            off_o[gs] = lax.shift_left(lax.bitwise_and(o, 1), 6)
            return 0
        lax.fori_loop(0, n_groups, idx_body, 0)

        for k in range(n_chunks):
            ks = pl.ds(k * _CHUNK, _CHUNK)
            cs_ = pltpu.async_copy(emb2_hbm.at[row_s.at[ks]], rows_s, sem)
            cr_ = pltpu.async_copy(rel2_hbm.at[row_r.at[ks]], rows_r, sem)
            co_ = pltpu.async_copy(emb2_hbm.at[row_o.at[ks]], rows_o, sem)
            cs_.wait()
            cr_.wait()
            co_.wait()

            def cgroup(g, _):
                gg = k * (_CHUNK // _LANES) + g
                gs = pl.ds(gg * _LANES, _LANES)
                ov_s = off_s[gs]
                ov_r = off_r[gs]
                ov_o = off_o[gs]
                for l in range(_LANES):
                    os_ = ov_s[l]
                    or_ = ov_r[l]
                    oo_ = ov_o[l]
                    i = g * _LANES + l
                    io = gg * _LANES + l
                    for c in range(D // _LANES):
                        out_v[io, pl.ds(c * _LANES, _LANES)] = (
                            rows_s[i, pl.ds(os_ + c * _LANES, _LANES)]
                            * rows_r[i, pl.ds(or_ + c * _LANES, _LANES)]
                            * rows_o[i, pl.ds(oo_ + c * _LANES, _LANES)])
                return 0
            lax.fori_loop(0, _CHUNK // _LANES, cgroup, 0)

        pltpu.sync_copy(out_v, out_hbm.at[pl.ds(base, b_per_w)])

    return dist_mult


def kernel(emb_e, emb_rel, triplets):
    B, D = triplets.shape[0], emb_e.shape[1]
    emb2 = emb_e.reshape(emb_e.shape[0] // 2, 2 * D)
    rel2 = emb_rel.reshape(emb_rel.shape[0] // 2, 2 * D)
    tflat = triplets.reshape(-1)
    fn = _build(B, D)
    return fn(tflat, emb2, rel2)


# speedup vs baseline: 1.0064x; 1.0064x over previous
import functools
import jax
import jax.numpy as jnp
from jax import lax
from jax.experimental import pallas as pl
from jax.experimental.pallas import tpu as pltpu
from jax.experimental.pallas import tpu_sc as plsc

_NUM_CORES = 2
_NUM_SUBCORES = 16
_NUM_WORKERS = _NUM_CORES * _NUM_SUBCORES
_LANES = 16
_CHUNK = 128


@functools.lru_cache(maxsize=None)
def _build(B, D):
    b_per_w = B // _NUM_WORKERS
    n_chunks = b_per_w // _CHUNK
    n_groups = b_per_w // _LANES
    mesh = plsc.VectorSubcoreMesh(
        core_axis_name="c", subcore_axis_name="s",
        num_cores=_NUM_CORES, num_subcores=_NUM_SUBCORES,
    )

    @functools.partial(
        pl.kernel,
        out_type=jax.ShapeDtypeStruct((B, D), jnp.float32),
        mesh=mesh,
        scratch_types=[
            pltpu.VMEM((b_per_w * 3,), jnp.int32),
            pltpu.VMEM((b_per_w,), jnp.int32),
            pltpu.VMEM((b_per_w,), jnp.int32),
            pltpu.VMEM((b_per_w,), jnp.int32),
            pltpu.VMEM((b_per_w,), jnp.int32),
            pltpu.VMEM((b_per_w,), jnp.int32),
            pltpu.VMEM((b_per_w,), jnp.int32),
            pltpu.VMEM((_CHUNK, 2 * D), jnp.float32),
            pltpu.VMEM((_CHUNK, 2 * D), jnp.float32),
            pltpu.VMEM((_CHUNK, 2 * D), jnp.float32),
            pltpu.VMEM((b_per_w, D), jnp.float32),
            pltpu.SemaphoreType.DMA,
        ],
        compiler_params=pltpu.CompilerParams(
            use_tc_tiling_on_sc=True, needs_layout_passes=False),
    )
    def dist_mult(tflat_hbm, emb2_hbm, rel2_hbm, out_hbm,
                  trip_v, row_s, row_r, row_o, off_s, off_r, off_o,
                  rows_s, rows_r, rows_o, out_v, sem):
        wid = lax.axis_index("s") * _NUM_CORES + lax.axis_index("c")
        base = wid * b_per_w

        pltpu.sync_copy(tflat_hbm.at[pl.ds(base * 3, b_per_w * 3)], trip_v)

        lanes3 = lax.iota(jnp.int32, 16) * 3

        def idx_body(g, _):
            offs = lanes3 + g * (3 * _LANES)
            gs = pl.ds(g * _LANES, _LANES)
            s = plsc.load_gather(trip_v, [offs])
            r = plsc.load_gather(trip_v, [offs + 1])
            o = plsc.load_gather(trip_v, [offs + 2])
            row_s[gs] = lax.shift_right_logical(s, 1)
            row_r[gs] = lax.shift_right_logical(r, 1)
            row_o[gs] = lax.shift_right_logical(o, 1)
            off_s[gs] = lax.shift_left(lax.bitwise_and(s, 1), 6)
            off_r[gs] = lax.shift_left(lax.bitwise_and(r, 1), 6)
            off_o[gs] = lax.shift_left(lax.bitwise_and(o, 1), 6)
            return 0
        lax.fori_loop(0, n_groups, idx_body, 0)

        for k in range(n_chunks):
            ks = pl.ds(k * _CHUNK, _CHUNK)
            cs_ = pltpu.async_copy(emb2_hbm.at[row_s.at[ks]], rows_s, sem)
            cr_ = pltpu.async_copy(rel2_hbm.at[row_r.at[ks]], rows_r, sem)
            co_ = pltpu.async_copy(emb2_hbm.at[row_o.at[ks]], rows_o, sem)
            cs_.wait()
            cr_.wait()
            co_.wait()

            def cgroup(g, _):
                gg = k * (_CHUNK // _LANES) + g
                gs = pl.ds(gg * _LANES, _LANES)
                ov_s = off_s[gs]
                ov_r = off_r[gs]
                ov_o = off_o[gs]
                for l in range(_LANES):
                    os_ = ov_s[l]
                    or_ = ov_r[l]
                    oo_ = ov_o[l]
                    i = g * _LANES + l
                    io = gg * _LANES + l
                    for c in range(D // _LANES):
                        out_v[io, pl.ds(c * _LANES, _LANES)] = (
                            rows_s[i, pl.ds(os_ + c * _LANES, _LANES)]
                            * rows_r[i, pl.ds(or_ + c * _LANES, _LANES)]
                            * rows_o[i, pl.ds(oo_ + c * _LANES, _LANES)])
                return 0
            lax.fori_loop(0, _CHUNK // _LANES, cgroup, 0)

        pltpu.sync_copy(out_v, out_hbm.at[pl.ds(base, b_per_w)])

    return dist_mult


def kernel(emb_e, emb_rel, triplets):
    B, D = triplets.shape[0], emb_e.shape[1]
    emb2 = emb_e.reshape(emb_e.shape[0] // 2, 2 * D)
    rel2 = emb_rel.reshape(emb_rel.shape[0] // 2, 2 * D)
    tflat = triplets.reshape(-1)
    fn = _build(B, D)
    return fn(tflat, emb2, rel2)


# speedup vs baseline: 1.9590x; 1.9464x over previous
import functools
import jax
import jax.numpy as jnp
from jax import lax
from jax.experimental import pallas as pl
from jax.experimental.pallas import tpu as pltpu
from jax.experimental.pallas import tpu_sc as plsc

_NUM_CORES = 2
_NUM_SUBCORES = 16
_NUM_WORKERS = _NUM_CORES * _NUM_SUBCORES
_LANES = 16
_CHUNK = 32


@functools.lru_cache(maxsize=None)
def _build(B, D):
    b_per_w = B // _NUM_WORKERS
    n_chunks = b_per_w // _CHUNK
    n_groups = b_per_w // _LANES
    mesh = plsc.VectorSubcoreMesh(
        core_axis_name="c", subcore_axis_name="s",
        num_cores=_NUM_CORES, num_subcores=_NUM_SUBCORES,
    )

    @functools.partial(
        pl.kernel,
        out_type=jax.ShapeDtypeStruct((B, D), jnp.float32),
        mesh=mesh,
        scratch_types=[
            pltpu.VMEM((b_per_w * 3,), jnp.int32),
            pltpu.VMEM((b_per_w,), jnp.int32),
            pltpu.VMEM((b_per_w,), jnp.int32),
            pltpu.VMEM((b_per_w,), jnp.int32),
            pltpu.VMEM((b_per_w,), jnp.int32),
            pltpu.VMEM((b_per_w,), jnp.int32),
            pltpu.VMEM((b_per_w,), jnp.int32),
            pltpu.VMEM((_CHUNK, 8, D), jnp.float32),
            pltpu.VMEM((_CHUNK, 8, D), jnp.float32),
            pltpu.VMEM((_CHUNK, 8, D), jnp.float32),
            pltpu.VMEM((_CHUNK, D), jnp.float32),
            pltpu.SemaphoreType.DMA,
        ],
        compiler_params=pltpu.CompilerParams(
            use_tc_tiling_on_sc=True, needs_layout_passes=False),
    )
    def dist_mult(tflat_hbm, emb8_hbm, rel8_hbm, out_hbm,
                  trip_v, blk_s, blk_r, blk_o, sub_s, sub_r, sub_o,
                  rows_s, rows_r, rows_o, out_v, sem):
        wid = lax.axis_index("s") * _NUM_CORES + lax.axis_index("c")
        base = wid * b_per_w

        pltpu.sync_copy(tflat_hbm.at[pl.ds(base * 3, b_per_w * 3)], trip_v)

        lanes3 = lax.iota(jnp.int32, 16) * 3

        def idx_body(g, _):
            offs = lanes3 + g * (3 * _LANES)
            gs = pl.ds(g * _LANES, _LANES)
            s = plsc.load_gather(trip_v, [offs])
            r = plsc.load_gather(trip_v, [offs + 1])
            o = plsc.load_gather(trip_v, [offs + 2])
            blk_s[gs] = lax.shift_right_logical(s, 3)
            blk_r[gs] = lax.shift_right_logical(r, 3)
            blk_o[gs] = lax.shift_right_logical(o, 3)
            sub_s[gs] = lax.bitwise_and(s, 7)
            sub_r[gs] = lax.bitwise_and(r, 7)
            sub_o[gs] = lax.bitwise_and(o, 7)
            return 0
        lax.fori_loop(0, n_groups, idx_body, 0)

        def chunk_body(k, _):
            def fire(g, _):
                gs = pl.ds(k * _CHUNK + g * _LANES, _LANES)
                bs = blk_s[gs]
                br = blk_r[gs]
                bo = blk_o[gs]
                for l in range(_LANES):
                    j = g * _LANES + l
                    pltpu.async_copy(
                        emb8_hbm.at[pl.ds(bs[l], 1)],
                        rows_s.at[pl.ds(j, 1)], sem)
                    pltpu.async_copy(
                        rel8_hbm.at[pl.ds(br[l], 1)],
                        rows_r.at[pl.ds(j, 1)], sem)
                    pltpu.async_copy(
                        emb8_hbm.at[pl.ds(bo[l], 1)],
                        rows_o.at[pl.ds(j, 1)], sem)
                return 0
            lax.fori_loop(0, _CHUNK // _LANES, fire, 0)
            pltpu.make_async_copy(
                emb8_hbm.at[pl.ds(0, _CHUNK)], rows_s, sem).wait()
            pltpu.make_async_copy(
                emb8_hbm.at[pl.ds(0, _CHUNK)], rows_r, sem).wait()
            pltpu.make_async_copy(
                emb8_hbm.at[pl.ds(0, _CHUNK)], rows_o, sem).wait()

            def cgroup(g, _):
                gs = pl.ds(k * _CHUNK + g * _LANES, _LANES)
                sv = sub_s[gs]
                rv = sub_r[gs]
                ov = sub_o[gs]
                for l in range(_LANES):
                    ss = sv[l]
                    sr = rv[l]
                    so = ov[l]
                    j = g * _LANES + l
                    for c in range(D // _LANES):
                        cs = pl.ds(c * _LANES, _LANES)
                        out_v[j, cs] = (rows_s[j, ss, cs]
                                        * rows_r[j, sr, cs]
                                        * rows_o[j, so, cs])
                return 0
            lax.fori_loop(0, _CHUNK // _LANES, cgroup, 0)

            pltpu.sync_copy(
                out_v, out_hbm.at[pl.ds(base + k * _CHUNK, _CHUNK)])
            return 0
        lax.fori_loop(0, n_chunks, chunk_body, 0)

    return dist_mult


def kernel(emb_e, emb_rel, triplets):
    B, D = triplets.shape[0], emb_e.shape[1]
    emb8 = emb_e.reshape(emb_e.shape[0] // 8, 8, D)
    rel8 = emb_rel.reshape(emb_rel.shape[0] // 8, 8, D)
    tflat = triplets.reshape(-1)
    fn = _build(B, D)
    return fn(tflat, emb8, rel8)


# speedup vs baseline: 2.0162x; 1.0292x over previous
import functools
import jax
import jax.numpy as jnp
from jax import lax
from jax.experimental import pallas as pl
from jax.experimental.pallas import tpu as pltpu
from jax.experimental.pallas import tpu_sc as plsc

_NUM_CORES = 2
_NUM_SUBCORES = 16
_NUM_WORKERS = _NUM_CORES * _NUM_SUBCORES
_LANES = 16
_CHUNK = 16


@functools.lru_cache(maxsize=None)
def _build(B, D, R):
    b_per_w = B // _NUM_WORKERS
    n_chunks = b_per_w // _CHUNK
    n_groups = b_per_w // _LANES
    mesh = plsc.VectorSubcoreMesh(
        core_axis_name="c", subcore_axis_name="s",
        num_cores=_NUM_CORES, num_subcores=_NUM_SUBCORES,
    )

    @functools.partial(
        pl.kernel,
        out_type=jax.ShapeDtypeStruct((B, D), jnp.float32),
        mesh=mesh,
        scratch_types=[
            pltpu.VMEM((b_per_w * 3,), jnp.int32),
            pltpu.VMEM((b_per_w,), jnp.int32),
            pltpu.VMEM((b_per_w,), jnp.int32),
            pltpu.VMEM((b_per_w,), jnp.int32),
            pltpu.VMEM((b_per_w,), jnp.int32),
            pltpu.VMEM((b_per_w,), jnp.int32),
            pltpu.VMEM((b_per_w,), jnp.int32),
            pltpu.VMEM((2, _CHUNK, 8, D), jnp.float32),
            pltpu.VMEM((2, _CHUNK, 8, D), jnp.float32),
            pltpu.VMEM((2, _CHUNK, 8, D), jnp.float32),
            pltpu.VMEM((_CHUNK, D), jnp.float32),
            pltpu.SemaphoreType.DMA,
        ],
        compiler_params=pltpu.CompilerParams(
            use_tc_tiling_on_sc=True, needs_layout_passes=False),
    )
    def dist_mult(tflat_hbm, emb8_hbm, rel8_hbm, out_hbm,
                  trip_v, blk_s, blk_r, blk_o, sub_s, sub_r, sub_o,
                  rows_s, rows_r, rows_o, out_v, sem):
        wid = lax.axis_index("s") * _NUM_CORES + lax.axis_index("c")
        base = wid * b_per_w

        pltpu.sync_copy(tflat_hbm.at[pl.ds(base * 3, b_per_w * 3)], trip_v)

        lanes3 = lax.iota(jnp.int32, 16) * 3

        def idx_body(g, _):
            offs = lanes3 + g * (3 * _LANES)
            gs = pl.ds(g * _LANES, _LANES)
            s = plsc.load_gather(trip_v, [offs])
            r = plsc.load_gather(trip_v, [offs + 1])
            o = plsc.load_gather(trip_v, [offs + 2])
            blk_s[gs] = lax.shift_right_logical(s, 3)
            blk_r[gs] = lax.shift_right_logical(r, 3)
            blk_o[gs] = lax.shift_right_logical(o, 3)
            sub_s[gs] = lax.bitwise_and(s, 7)
            sub_r[gs] = lax.bitwise_and(r, 7)
            sub_o[gs] = lax.bitwise_and(o, 7)
            return 0
        lax.fori_loop(0, n_groups, idx_body, 0)

        def fire(kk):
            p = lax.rem(kk, 2)
            gs = pl.ds(kk * _CHUNK, _CHUNK)
            bs = blk_s[gs]
            br = blk_r[gs]
            bo = blk_o[gs]
            for l in range(_CHUNK):
                pltpu.async_copy(
                    emb8_hbm.at[pl.ds(bs[l], 1)],
                    rows_s.at[p].at[pl.ds(l, 1)], sem)
                pltpu.async_copy(
                    rel8_hbm.at[pl.ds(br[l], 1)],
                    rows_r.at[p].at[pl.ds(l, 1)], sem)
                pltpu.async_copy(
                    emb8_hbm.at[pl.ds(bo[l], 1)],
                    rows_o.at[p].at[pl.ds(l, 1)], sem)

        fire(jnp.int32(0))

        def chunk_body(k, _):
            @pl.when(k + 1 < n_chunks)
            def _():
                fire(k + 1)

            p = lax.rem(k, 2)
            pltpu.make_async_copy(
                emb8_hbm.at[pl.ds(0, _CHUNK)], rows_s.at[0], sem).wait()
            pltpu.make_async_copy(
                emb8_hbm.at[pl.ds(0, _CHUNK)], rows_r.at[0], sem).wait()
            pltpu.make_async_copy(
                emb8_hbm.at[pl.ds(0, _CHUNK)], rows_o.at[0], sem).wait()

            gs = pl.ds(k * _CHUNK, _CHUNK)
            sv = sub_s[gs]
            rv = sub_r[gs]
            ov = sub_o[gs]
            for l in range(_CHUNK):
                ss = sv[l]
                sr = rv[l]
                so = ov[l]
                for c in range(D // _LANES):
                    cs = pl.ds(c * _LANES, _LANES)
                    out_v[l, cs] = (rows_s[p, l, ss, cs]
                                    * rows_r[p, l, sr, cs]
                                    * rows_o[p, l, so, cs])

            pltpu.sync_copy(
                out_v, out_hbm.at[pl.ds(base + k * _CHUNK, _CHUNK)])
            return 0
        lax.fori_loop(0, n_chunks, chunk_body, 0)

    return dist_mult


def kernel(emb_e, emb_rel, triplets):
    B, D = triplets.shape[0], emb_e.shape[1]
    R = emb_rel.shape[0]
    emb8 = emb_e.reshape(emb_e.shape[0] // 8, 8, D)
    rel8 = emb_rel.reshape(R // 8, 8, D)
    tflat = triplets.reshape(-1)
    fn = _build(B, D, R)
    return fn(tflat, emb8, rel8)


# speedup vs baseline: 2.0166x; 1.0002x over previous
import functools
import jax
import jax.numpy as jnp
from jax import lax
from jax.experimental import pallas as pl
from jax.experimental.pallas import tpu as pltpu
from jax.experimental.pallas import tpu_sc as plsc

_NUM_CORES = 2
_NUM_SUBCORES = 16
_NUM_WORKERS = _NUM_CORES * _NUM_SUBCORES
_LANES = 16
_CHUNK = 16


@functools.lru_cache(maxsize=None)
def _build(B, D, R):
    b_per_w = B // _NUM_WORKERS
    n_chunks = b_per_w // _CHUNK
    n_groups = b_per_w // _LANES
    mesh = plsc.VectorSubcoreMesh(
        core_axis_name="c", subcore_axis_name="s",
        num_cores=_NUM_CORES, num_subcores=_NUM_SUBCORES,
    )

    @functools.partial(
        pl.kernel,
        out_type=jax.ShapeDtypeStruct((B, D), jnp.float32),
        mesh=mesh,
        scratch_types=[
            pltpu.VMEM((b_per_w * 3,), jnp.int32),
            pltpu.VMEM((b_per_w,), jnp.int32),
            pltpu.VMEM((b_per_w,), jnp.int32),
            pltpu.VMEM((b_per_w,), jnp.int32),
            pltpu.VMEM((b_per_w,), jnp.int32),
            pltpu.VMEM((b_per_w,), jnp.int32),
            pltpu.VMEM((b_per_w,), jnp.int32),
            pltpu.VMEM((2, _CHUNK, 8, D), jnp.float32),
            pltpu.VMEM((2, _CHUNK, 8, D), jnp.float32),
            pltpu.VMEM((2, _CHUNK, 8, D), jnp.float32),
            pltpu.VMEM((2, _CHUNK, D), jnp.float32),
            pltpu.SemaphoreType.DMA,
            pltpu.SemaphoreType.DMA,
        ],
        compiler_params=pltpu.CompilerParams(
            use_tc_tiling_on_sc=True, needs_layout_passes=False),
    )
    def dist_mult(tflat_hbm, emb8_hbm, rel8_hbm, out_hbm,
                  trip_v, blk_s, blk_r, blk_o, sub_s, sub_r, sub_o,
                  rows_s, rows_r, rows_o, out_v, sem, osem):
        wid = lax.axis_index("s") * _NUM_CORES + lax.axis_index("c")
        base = wid * b_per_w

        pltpu.sync_copy(tflat_hbm.at[pl.ds(base * 3, b_per_w * 3)], trip_v)

        lanes3 = lax.iota(jnp.int32, 16) * 3

        def idx_body(g, _):
            offs = lanes3 + g * (3 * _LANES)
            gs = pl.ds(g * _LANES, _LANES)
            s = plsc.load_gather(trip_v, [offs])
            r = plsc.load_gather(trip_v, [offs + 1])
            o = plsc.load_gather(trip_v, [offs + 2])
            blk_s[gs] = lax.shift_right_logical(s, 3)
            blk_r[gs] = lax.shift_right_logical(r, 3)
            blk_o[gs] = lax.shift_right_logical(o, 3)
            sub_s[gs] = lax.bitwise_and(s, 7)
            sub_r[gs] = lax.bitwise_and(r, 7)
            sub_o[gs] = lax.bitwise_and(o, 7)
            return 0
        lax.fori_loop(0, n_groups, idx_body, 0)

        def fire(kk):
            p = lax.rem(kk, 2)
            gs = pl.ds(kk * _CHUNK, _CHUNK)
            bs = blk_s[gs]
            br = blk_r[gs]
            bo = blk_o[gs]
            for l in range(_CHUNK):
                pltpu.async_copy(
                    emb8_hbm.at[pl.ds(bs[l], 1)],
                    rows_s.at[p].at[pl.ds(l, 1)], sem)
                pltpu.async_copy(
                    rel8_hbm.at[pl.ds(br[l], 1)],
                    rows_r.at[p].at[pl.ds(l, 1)], sem)
                pltpu.async_copy(
                    emb8_hbm.at[pl.ds(bo[l], 1)],
                    rows_o.at[p].at[pl.ds(l, 1)], sem)

        fire(jnp.int32(0))

        def chunk_body(k, _):
            @pl.when(k + 1 < n_chunks)
            def _():
                fire(k + 1)

            p = lax.rem(k, 2)
            pltpu.make_async_copy(
                emb8_hbm.at[pl.ds(0, _CHUNK)], rows_s.at[0], sem).wait()
            pltpu.make_async_copy(
                emb8_hbm.at[pl.ds(0, _CHUNK)], rows_r.at[0], sem).wait()
            pltpu.make_async_copy(
                emb8_hbm.at[pl.ds(0, _CHUNK)], rows_o.at[0], sem).wait()

            gs = pl.ds(k * _CHUNK, _CHUNK)
            sv = sub_s[gs]
            rv = sub_r[gs]
            ov = sub_o[gs]
            for l in range(_CHUNK):
                ss = sv[l]
                sr = rv[l]
                so = ov[l]
                for c in range(D // _LANES):
                    cs = pl.ds(c * _LANES, _LANES)
                    out_v[p, l, cs] = (rows_s[p, l, ss, cs]
                                       * rows_r[p, l, sr, cs]
                                       * rows_o[p, l, so, cs])

            pltpu.async_copy(
                out_v.at[p],
                out_hbm.at[pl.ds(base + k * _CHUNK, _CHUNK)], osem)
            @pl.when(k >= 2)
            def _():
                pltpu.make_async_copy(
                    out_hbm.at[pl.ds(base, _CHUNK)], out_v.at[0],
                    osem).wait()
            return 0
        lax.fori_loop(0, n_chunks, chunk_body, 0)

        pltpu.make_async_copy(
            out_hbm.at[pl.ds(base, _CHUNK)], out_v.at[0], osem).wait()
        pltpu.make_async_copy(
            out_hbm.at[pl.ds(base, _CHUNK)], out_v.at[0], osem).wait()

    return dist_mult


def kernel(emb_e, emb_rel, triplets):
    B, D = triplets.shape[0], emb_e.shape[1]
    R = emb_rel.shape[0]
    emb8 = emb_e.reshape(emb_e.shape[0] // 8, 8, D)
    rel8 = emb_rel.reshape(R // 8, 8, D)
    tflat = triplets.reshape(-1)
    fn = _build(B, D, R)
    return fn(tflat, emb8, rel8)
